# baseline (speedup 1.0000x reference)
import jax
import jax.numpy as jnp
from jax import lax
from jax.experimental import pallas as pl
from jax.experimental.pallas import tpu as pltpu
from jax.experimental.pallas import tpu_sc as plsc

S = 2048
H = 1024
FH = 4 * H
E = 8
K = 2
P = K * S
CAP = S
TM = 256
MAXT = CAP // TM
NB = 8
BS = P // NB
MT = 24

_NC, _NS = 2, 16
NW = _NC * _NS
PPW = P // NW
CHUNK = 64


def _routing_body(x_ref, wg_ref, pos_ref, w_ref, meta_ref):
    logits = jnp.dot(x_ref[...], wg_ref[...], preferred_element_type=jnp.float32)
    m = jnp.max(logits, axis=-1, keepdims=True)
    ex = jnp.exp(logits - m)
    probs = ex / jnp.sum(ex, axis=-1, keepdims=True)

    iota = lax.broadcasted_iota(jnp.int32, (S, E), 1)
    v0 = jnp.max(probs, axis=-1, keepdims=True)
    i0 = jnp.min(jnp.where(probs == v0, iota, E), axis=-1, keepdims=True)
    probs1 = jnp.where(iota == i0, -1.0, probs)
    v1 = jnp.max(probs1, axis=-1, keepdims=True)
    i1 = jnp.min(jnp.where(probs1 == v1, iota, E), axis=-1, keepdims=True)

    oh = jnp.concatenate(
        [(iota == i0).astype(jnp.float32), (iota == i1).astype(jnp.float32)],
        axis=0)
    e_all = jnp.concatenate([i0, i1], axis=0)

    tri = (lax.broadcasted_iota(jnp.int32, (BS, BS), 0)
           > lax.broadcasted_iota(jnp.int32, (BS, BS), 1)).astype(jnp.float32)
    running = jnp.zeros((1, E), dtype=jnp.float32)
    for b in range(NB):
        oh_b = oh[b * BS:(b + 1) * BS]
        partial = jnp.dot(tri, oh_b, preferred_element_type=jnp.float32)
        rank_b = partial + running
        rank_at = jnp.sum(rank_b * oh_b, axis=-1, keepdims=True)
        e_b = e_all[b * BS:(b + 1) * BS]
        pos_ref[b * BS:(b + 1) * BS, :] = e_b * CAP + rank_at.astype(jnp.int32)
        running = running + jnp.sum(oh_b, axis=0, keepdims=True)

    w_ref[...] = jnp.concatenate([v0, v1], axis=1)

    nt_f = jnp.ceil(running * (1.0 / TM))
    tri_e = (lax.broadcasted_iota(jnp.int32, (E, E), 0)
             < lax.broadcasted_iota(jnp.int32, (E, E), 1)).astype(jnp.float32)
    cum = jnp.dot(nt_f, tri_e, preferred_element_type=jnp.float32)
    nv = jnp.sum(nt_f, axis=-1, keepdims=True)
    iota_t = lax.broadcasted_iota(jnp.int32, (MT, 1), 0).astype(jnp.float32)
    clamped = jnp.minimum(iota_t, nv - 1.0)
    onehot = ((clamped >= cum) & (clamped < cum + nt_f)).astype(jnp.float32)
    iota_e = lax.broadcasted_iota(jnp.int32, (1, E), 1).astype(jnp.float32)
    tg = jnp.sum(onehot * iota_e, axis=-1, keepdims=True)
    jl = clamped - jnp.sum(onehot * cum, axis=-1, keepdims=True)
    tr = tg * MAXT + jl
    nv_col = jnp.broadcast_to(nv, (MT, 1))
    meta_ref[...] = jnp.concatenate(
        [tg, tr, nv_col, nv_col], axis=1).astype(jnp.int32)


def _routing(x2, Wg):
    return pl.pallas_call(
        _routing_body,
        out_shape=(
            jax.ShapeDtypeStruct((P, 1), jnp.int32),
            jax.ShapeDtypeStruct((S, K), jnp.float32),
            jax.ShapeDtypeStruct((MT, 4), jnp.int32),
        ),
    )(x2, Wg)


def _worker_id():
    return lax.axis_index("s") * _NC + lax.axis_index("c")


TPW = S // NW


def _sc_scatter_body(xb_hbm, pos_hbm, xcap_hbm, pos0_v, pos1_v, rows_v, sem):
    wid = _worker_id()
    t0 = wid * TPW
    pltpu.sync_copy(pos_hbm.at[pl.ds(t0, TPW)], pos0_v)
    pltpu.sync_copy(pos_hbm.at[pl.ds(S + t0, TPW)], pos1_v)
    pltpu.sync_copy(xb_hbm.at[pl.ds(t0, TPW)], rows_v)
    c0 = pltpu.async_copy(rows_v, xcap_hbm.at[pos0_v], sem)
    c1 = pltpu.async_copy(rows_v, xcap_hbm.at[pos1_v], sem)
    c0.wait()
    c1.wait()


def _sc_scatter(xb3, pos):
    mesh = plsc.VectorSubcoreMesh(core_axis_name="c", subcore_axis_name="s")
    return pl.kernel(
        _sc_scatter_body,
        out_type=jax.ShapeDtypeStruct((E * CAP, H // 2), jnp.int32),
        mesh=mesh,
        scratch_types=[
            pltpu.VMEM((TPW,), jnp.int32),
            pltpu.VMEM((TPW,), jnp.int32),
            pltpu.VMEM((TPW, H // 2), jnp.int32),
            pltpu.SemaphoreType.DMA,
        ],
    )(xb3, pos)


def _gelu_exact(h):
    return 0.5 * h * (1.0 + lax.erf(h * (2.0 ** -0.5)))


def _ffn_body(meta_ref, x_ref, w1_ref, b1_ref, w2_ref, b2_ref, y_ref):
    i = pl.program_id(0)

    @pl.when(i < meta_ref[0, 2])
    def _():
        xb = x_ref[...]
        h = jnp.dot(xb, w1_ref[0], preferred_element_type=jnp.float32)
        h = _gelu_exact(h + b1_ref[0]).astype(jnp.bfloat16)
        y_ref[...] = (jnp.dot(h, w2_ref[0], preferred_element_type=jnp.float32)
                      + b2_ref[0])


def _ffn(meta, x_cap, W1, b1, W2, b2):
    grid_spec = pltpu.PrefetchScalarGridSpec(
        num_scalar_prefetch=1,
        grid=(MT,),
        in_specs=[
            pl.BlockSpec((TM, H), lambda i, m: (m[i, 1], 0)),
            pl.BlockSpec((1, H, FH), lambda i, m: (m[i, 0], 0, 0)),
            pl.BlockSpec((1, 1, FH), lambda i, m: (m[i, 0], 0, 0)),
            pl.BlockSpec((1, FH, H), lambda i, m: (m[i, 0], 0, 0)),
            pl.BlockSpec((1, 1, H), lambda i, m: (m[i, 0], 0, 0)),
        ],
        out_specs=pl.BlockSpec((TM, H), lambda i, m: (m[i, 1], 0)),
    )
    return pl.pallas_call(
        _ffn_body,
        grid_spec=grid_spec,
        out_shape=jax.ShapeDtypeStruct((E * CAP, H), jnp.float32),
        compiler_params=pltpu.CompilerParams(
            dimension_semantics=("arbitrary",)),
    )(meta, x_cap, W1.astype(jnp.bfloat16), b1.reshape(E, 1, FH),
      W2.astype(jnp.bfloat16), b2.reshape(E, 1, H))


def _sc_gather_body(ycap_hbm, pos_hbm, out_hbm, pos_v, rows_v, sem):
    wid = _worker_id()
    for c in range(PPW // CHUNK):
        base = wid * PPW + c * CHUNK
        pltpu.sync_copy(pos_hbm.at[pl.ds(base, CHUNK)], pos_v)
        pltpu.async_copy(ycap_hbm.at[pos_v], rows_v, sem).wait()
        pltpu.sync_copy(rows_v, out_hbm.at[pl.ds(base, CHUNK)])


def _sc_gather(y_cap, pos):
    mesh = plsc.VectorSubcoreMesh(core_axis_name="c", subcore_axis_name="s")
    return pl.kernel(
        _sc_gather_body,
        out_type=jax.ShapeDtypeStruct((P, H), jnp.float32),
        mesh=mesh,
        scratch_types=[
            pltpu.VMEM((CHUNK,), jnp.int32),
            pltpu.VMEM((CHUNK, H), jnp.float32),
            pltpu.SemaphoreType.DMA,
        ],
    )(y_cap, pos)


def _combine_body(y0_ref, y1_ref, w_ref, o_ref):
    w = w_ref[...]
    o_ref[...] = y0_ref[...] * w[:, 0:1] + y1_ref[...] * w[:, 1:2]


def _combine(y01, wT):
    bt = 256
    return pl.pallas_call(
        _combine_body,
        grid=(S // bt,),
        in_specs=[
            pl.BlockSpec((bt, H), lambda t: (t, 0)),
            pl.BlockSpec((bt, H), lambda t: (t + S // bt, 0)),
            pl.BlockSpec((bt, K), lambda t: (t, 0)),
        ],
        out_specs=pl.BlockSpec((bt, H), lambda t: (t, 0)),
        out_shape=jax.ShapeDtypeStruct((S, H), jnp.float32),
    )(y01, y01, wT)


def kernel(x, Wg, W1, b1, W2, b2):
    x2 = x.reshape(S, H)
    pos2, wT, meta = _routing(x2, Wg)
    pos = pos2.reshape(P)
    xi = lax.bitcast_convert_type(
        x2.astype(jnp.bfloat16).reshape(S, H // 2, 2), jnp.int32)
    x_cap = lax.bitcast_convert_type(
        _sc_scatter(xi, pos), jnp.bfloat16).reshape(E * CAP, H)
    y_cap = _ffn(meta, x_cap, W1, b1, W2, b2)
    y01 = _sc_gather(y_cap, pos)
    out = _combine(y01, wT)
    return out.reshape(1, S, H)


# speedup vs baseline: 1.9164x; 1.9164x over previous
import jax
import jax.numpy as jnp
from jax import lax
from jax.experimental import pallas as pl
from jax.experimental.pallas import tpu as pltpu
from jax.experimental.pallas import tpu_sc as plsc

S = 2048
H = 1024
FH = 4 * H
E = 8
K = 2
P = K * S
CAP = S
TM = 256
MAXT = CAP // TM
NB = 8
BS = P // NB
MT = 24

_NC, _NS = 2, 16
NW = _NC * _NS
PPW = P // NW
CHUNK = 64


def _routing_body(x_ref, wg_ref, pos_ref, w_ref, meta_ref):
    logits = jnp.dot(x_ref[...], wg_ref[...], preferred_element_type=jnp.float32)
    m = jnp.max(logits, axis=-1, keepdims=True)
    ex = jnp.exp(logits - m)
    probs = ex / jnp.sum(ex, axis=-1, keepdims=True)

    iota = lax.broadcasted_iota(jnp.int32, (S, E), 1)
    v0 = jnp.max(probs, axis=-1, keepdims=True)
    i0 = jnp.min(jnp.where(probs == v0, iota, E), axis=-1, keepdims=True)
    probs1 = jnp.where(iota == i0, -1.0, probs)
    v1 = jnp.max(probs1, axis=-1, keepdims=True)
    i1 = jnp.min(jnp.where(probs1 == v1, iota, E), axis=-1, keepdims=True)

    oh = jnp.concatenate(
        [(iota == i0).astype(jnp.float32), (iota == i1).astype(jnp.float32)],
        axis=0)
    e_all = jnp.concatenate([i0, i1], axis=0)

    tri = (lax.broadcasted_iota(jnp.int32, (BS, BS), 0)
           > lax.broadcasted_iota(jnp.int32, (BS, BS), 1)).astype(jnp.float32)
    running = jnp.zeros((1, E), dtype=jnp.float32)
    for b in range(NB):
        oh_b = oh[b * BS:(b + 1) * BS]
        partial = jnp.dot(tri, oh_b, preferred_element_type=jnp.float32)
        rank_b = partial + running
        rank_at = jnp.sum(rank_b * oh_b, axis=-1, keepdims=True)
        e_b = e_all[b * BS:(b + 1) * BS]
        pos_ref[b * BS:(b + 1) * BS, :] = e_b * CAP + rank_at.astype(jnp.int32)
        running = running + jnp.sum(oh_b, axis=0, keepdims=True)

    w_ref[...] = jnp.concatenate([v0, v1], axis=1)

    nt_f = jnp.ceil(running * (1.0 / TM))
    tri_e = (lax.broadcasted_iota(jnp.int32, (E, E), 0)
             < lax.broadcasted_iota(jnp.int32, (E, E), 1)).astype(jnp.float32)
    cum = jnp.dot(nt_f, tri_e, preferred_element_type=jnp.float32)
    nv = jnp.sum(nt_f, axis=-1, keepdims=True)
    iota_t = lax.broadcasted_iota(jnp.int32, (MT, 1), 0).astype(jnp.float32)
    clamped = jnp.minimum(iota_t, nv - 1.0)
    onehot = ((clamped >= cum) & (clamped < cum + nt_f)).astype(jnp.float32)
    iota_e = lax.broadcasted_iota(jnp.int32, (1, E), 1).astype(jnp.float32)
    tg = jnp.sum(onehot * iota_e, axis=-1, keepdims=True)
    jl = clamped - jnp.sum(onehot * cum, axis=-1, keepdims=True)
    tr = tg * MAXT + jl
    nv_col = jnp.broadcast_to(nv, (MT, 1))
    meta_ref[...] = jnp.concatenate(
        [tg, tr, nv_col, nv_col], axis=1).astype(jnp.int32)


def _routing(x2, Wg):
    return pl.pallas_call(
        _routing_body,
        out_shape=(
            jax.ShapeDtypeStruct((P, 1), jnp.int32),
            jax.ShapeDtypeStruct((S, K), jnp.float32),
            jax.ShapeDtypeStruct((MT, 4), jnp.int32),
        ),
    )(x2, Wg)


def _worker_id():
    return lax.axis_index("s") * _NC + lax.axis_index("c")


TPW = S // NW


def _sc_scatter_body(xb_hbm, pos_hbm, xcap_hbm, pos0_v, pos1_v, rows_v, sem):
    wid = _worker_id()
    t0 = wid * TPW
    pltpu.sync_copy(pos_hbm.at[pl.ds(t0, TPW)], pos0_v)
    pltpu.sync_copy(pos_hbm.at[pl.ds(S + t0, TPW)], pos1_v)
    pltpu.sync_copy(xb_hbm.at[pl.ds(t0, TPW)], rows_v)
    c0 = pltpu.async_copy(rows_v, xcap_hbm.at[pos0_v], sem)
    c1 = pltpu.async_copy(rows_v, xcap_hbm.at[pos1_v], sem)
    c0.wait()
    c1.wait()


def _sc_scatter(x2, pos):
    mesh = plsc.VectorSubcoreMesh(core_axis_name="c", subcore_axis_name="s")
    return pl.kernel(
        _sc_scatter_body,
        out_type=jax.ShapeDtypeStruct((E * CAP, H), jnp.float32),
        mesh=mesh,
        scratch_types=[
            pltpu.VMEM((TPW,), jnp.int32),
            pltpu.VMEM((TPW,), jnp.int32),
            pltpu.VMEM((TPW, H), jnp.float32),
            pltpu.SemaphoreType.DMA,
        ],
    )(x2, pos)


def _gelu_exact(h):
    return 0.5 * h * (1.0 + lax.erf(h * (2.0 ** -0.5)))


def _ffn_body(meta_ref, x_ref, w1_ref, b1_ref, w2_ref, b2_ref, y_ref):
    i = pl.program_id(0)

    @pl.when(i < meta_ref[0, 2])
    def _():
        xb = x_ref[...].astype(jnp.bfloat16)
        h = jnp.dot(xb, w1_ref[0], preferred_element_type=jnp.float32)
        h = _gelu_exact(h + b1_ref[0]).astype(jnp.bfloat16)
        y_ref[...] = (jnp.dot(h, w2_ref[0], preferred_element_type=jnp.float32)
                      + b2_ref[0])


def _ffn(meta, x_cap, W1, b1, W2, b2):
    grid_spec = pltpu.PrefetchScalarGridSpec(
        num_scalar_prefetch=1,
        grid=(MT,),
        in_specs=[
            pl.BlockSpec((TM, H), lambda i, m: (m[i, 1], 0)),
            pl.BlockSpec((1, H, FH), lambda i, m: (m[i, 0], 0, 0)),
            pl.BlockSpec((1, 1, FH), lambda i, m: (m[i, 0], 0, 0)),
            pl.BlockSpec((1, FH, H), lambda i, m: (m[i, 0], 0, 0)),
            pl.BlockSpec((1, 1, H), lambda i, m: (m[i, 0], 0, 0)),
        ],
        out_specs=pl.BlockSpec((TM, H), lambda i, m: (m[i, 1], 0)),
    )
    return pl.pallas_call(
        _ffn_body,
        grid_spec=grid_spec,
        out_shape=jax.ShapeDtypeStruct((E * CAP, H), jnp.float32),
        compiler_params=pltpu.CompilerParams(
            dimension_semantics=("arbitrary",)),
    )(meta, x_cap, W1.astype(jnp.bfloat16), b1.reshape(E, 1, FH),
      W2.astype(jnp.bfloat16), b2.reshape(E, 1, H))


def _sc_gather_body(ycap_hbm, pos_hbm, out_hbm, pos_v, rows_v, sem):
    wid = _worker_id()
    for c in range(PPW // CHUNK):
        base = wid * PPW + c * CHUNK
        pltpu.sync_copy(pos_hbm.at[pl.ds(base, CHUNK)], pos_v)
        pltpu.async_copy(ycap_hbm.at[pos_v], rows_v, sem).wait()
        pltpu.sync_copy(rows_v, out_hbm.at[pl.ds(base, CHUNK)])


def _sc_gather(y_cap, pos):
    mesh = plsc.VectorSubcoreMesh(core_axis_name="c", subcore_axis_name="s")
    return pl.kernel(
        _sc_gather_body,
        out_type=jax.ShapeDtypeStruct((P, H), jnp.float32),
        mesh=mesh,
        scratch_types=[
            pltpu.VMEM((CHUNK,), jnp.int32),
            pltpu.VMEM((CHUNK, H), jnp.float32),
            pltpu.SemaphoreType.DMA,
        ],
    )(y_cap, pos)


def _combine_body(y0_ref, y1_ref, w_ref, o_ref):
    w = w_ref[...]
    o_ref[...] = y0_ref[...] * w[:, 0:1] + y1_ref[...] * w[:, 1:2]


def _combine(y01, wT):
    bt = 256
    return pl.pallas_call(
        _combine_body,
        grid=(S // bt,),
        in_specs=[
            pl.BlockSpec((bt, H), lambda t: (t, 0)),
            pl.BlockSpec((bt, H), lambda t: (t + S // bt, 0)),
            pl.BlockSpec((bt, K), lambda t: (t, 0)),
        ],
        out_specs=pl.BlockSpec((bt, H), lambda t: (t, 0)),
        out_shape=jax.ShapeDtypeStruct((S, H), jnp.float32),
    )(y01, y01, wT)


def kernel(x, Wg, W1, b1, W2, b2):
    x2 = x.reshape(S, H)
    pos2, wT, meta = _routing(x2, Wg)
    pos = pos2.reshape(P)
    x_cap = _sc_scatter(x2, pos)
    y_cap = _ffn(meta, x_cap, W1, b1, W2, b2)
    y01 = _sc_gather(y_cap, pos)
    out = _combine(y01, wT)
    return out.reshape(1, S, H)
